# Initial kernel scaffold; baseline (speedup 1.0000x reference)
import jax
import jax.numpy as jnp
from jax import lax
from jax.experimental import pallas as pl
from jax.experimental.pallas import tpu as pltpu
from jax.experimental.pallas import tpu_sc as plsc

N = 10000
E = 320000
D = 128

NC = 2
NS = 16
NW = NC * NS
B = 128
NBATCH = (E + NW * B - 1) // (NW * B)
E_PAD = NW * B * NBATCH
N_PAD = 10240
ROWS_PER_TILE_PAD = N_PAD // NS
ROWS_PER_TILE = N // NS
DEG_W = 16

_mesh = plsc.VectorSubcoreMesh(core_axis_name="c", subcore_axis_name="s")


def _fill_vmem(ref, rows, width, value):
    vv = jnp.full((16,), value, jnp.float32)

    @pl.loop(0, rows)
    def _(i):
        @pl.loop(0, width // 16)
        def _(j):
            ref[i, pl.ds(j * 16, 16)] = vv


def _deg_body(dst_hbm, deg_hbm, dstb, onesb, zb, deg_sh, sem):
    c = lax.axis_index("c")
    s = lax.axis_index("s")
    w = s * NC + c

    _fill_vmem(zb, B, DEG_W, 0.0)
    @pl.loop(0, ROWS_PER_TILE_PAD // B)
    def _(k):
        pltpu.sync_copy(zb, deg_sh.at[pl.ds(s * ROWS_PER_TILE_PAD + k * B, B)])
    _fill_vmem(onesb, B, DEG_W, 1.0)

    pltpu.async_copy(dst_hbm.at[w], dstb, sem).wait()
    plsc.subcore_barrier()

    @pl.loop(0, NBATCH)
    def _(j):
        pltpu.sync_copy(onesb, deg_sh.at[dstb.at[j]], add=True)

    plsc.subcore_barrier()
    pltpu.sync_copy(deg_sh.at[pl.ds(s * ROWS_PER_TILE, ROWS_PER_TILE)],
                    deg_hbm.at[pl.ds(c * N + s * ROWS_PER_TILE, ROWS_PER_TILE)])


@jax.jit
def _deg(dst_r):
    return pl.kernel(
        _deg_body,
        out_type=jax.ShapeDtypeStruct((NC * N, DEG_W), jnp.float32),
        mesh=_mesh,
        scratch_types=[
            pltpu.VMEM((NBATCH, B), jnp.int32),
            pltpu.VMEM((B, DEG_W), jnp.float32),
            pltpu.VMEM((B, DEG_W), jnp.float32),
            pltpu.VMEM_SHARED((N_PAD, DEG_W), jnp.float32),
            pltpu.SemaphoreType.DMA,
        ],
    )(dst_r)


def _scatter_body(g_hbm, src_hbm, dst_hbm, out_hbm, srcb, dstb, rows, zrows,
                  acc_sh, sem):
    c = lax.axis_index("c")
    s = lax.axis_index("s")
    w = s * NC + c

    _fill_vmem(zrows, B, D, 0.0)
    @pl.loop(0, ROWS_PER_TILE_PAD // B)
    def _(k):
        pltpu.sync_copy(zrows, acc_sh.at[pl.ds(s * ROWS_PER_TILE_PAD + k * B, B)])

    pltpu.async_copy(src_hbm.at[w], srcb, sem).wait()
    pltpu.async_copy(dst_hbm.at[w], dstb, sem).wait()
    plsc.subcore_barrier()

    @pl.loop(0, NBATCH)
    def _(j):
        pltpu.async_copy(g_hbm.at[srcb.at[j]], rows, sem).wait()
        pltpu.sync_copy(rows, acc_sh.at[dstb.at[j]], add=True)

    plsc.subcore_barrier()
    pltpu.sync_copy(acc_sh.at[pl.ds(s * ROWS_PER_TILE, ROWS_PER_TILE)],
                    out_hbm.at[pl.ds(c * N + s * ROWS_PER_TILE, ROWS_PER_TILE)])


@jax.jit
def _scatter(g, src_r, dst_r):
    return pl.kernel(
        _scatter_body,
        out_type=jax.ShapeDtypeStruct((NC * N, D), jnp.float32),
        mesh=_mesh,
        scratch_types=[
            pltpu.VMEM((NBATCH, B), jnp.int32),
            pltpu.VMEM((NBATCH, B), jnp.int32),
            pltpu.VMEM((B, D), jnp.float32),
            pltpu.VMEM((B, D), jnp.float32),
            pltpu.VMEM_SHARED((N_PAD, D), jnp.float32),
            pltpu.SemaphoreType.DMA,
        ],
    )(g, src_r, dst_r)


def _dis_col(deg_ref):
    deg = (deg_ref[0:N, 0:1] + deg_ref[N:2 * N, 0:1]) + 1.0
    return lax.rsqrt(deg)


def _mm1_body(x_ref, w_ref, deg_ref, g_ref):
    dis = _dis_col(deg_ref)
    h = jnp.dot(x_ref[...], w_ref[...], preferred_element_type=jnp.float32)
    g_ref[0:N, :] = dis * h
    g_ref[N:N_PAD, :] = jnp.zeros((N_PAD - N, D), jnp.float32)


@jax.jit
def _mm1(x, W1, deg_p):
    return pl.pallas_call(
        _mm1_body,
        out_shape=jax.ShapeDtypeStruct((N_PAD, D), jnp.float32),
    )(x, W1, deg_p)


def _mm2_body(acc_ref, g1_ref, deg_ref, b_ref, w_ref, g2_ref):
    dis = _dis_col(deg_ref)
    acc = acc_ref[0:N, :] + acc_ref[N:2 * N, :]
    h1 = jnp.maximum(dis * (acc + g1_ref[0:N, :]) + b_ref[...], 0.0)
    h2 = jnp.dot(h1, w_ref[...], preferred_element_type=jnp.float32)
    g2_ref[0:N, :] = dis * h2
    g2_ref[N:N_PAD, :] = jnp.zeros((N_PAD - N, D), jnp.float32)


@jax.jit
def _mm2(acc1, g1, deg_p, b1, W2):
    return pl.pallas_call(
        _mm2_body,
        out_shape=jax.ShapeDtypeStruct((N_PAD, D), jnp.float32),
    )(acc1, g1, deg_p, b1, W2)


def _fin_body(acc_ref, g2_ref, deg_ref, b_ref, out_ref):
    dis = _dis_col(deg_ref)
    acc = acc_ref[0:N, :] + acc_ref[N:2 * N, :]
    out_ref[...] = dis * (acc + g2_ref[0:N, :]) + b_ref[...]


@jax.jit
def _fin(acc2, g2, deg_p, b2):
    return pl.pallas_call(
        _fin_body,
        out_shape=jax.ShapeDtypeStruct((N, D), jnp.float32),
    )(acc2, g2, deg_p, b2)


def kernel(x, edge_index, W1, b1, W2, b2):
    src = edge_index[0].astype(jnp.int32)
    dst = edge_index[1].astype(jnp.int32)
    pad = E_PAD - E
    pad_idx = N + (jnp.arange(pad, dtype=jnp.int32) % (N_PAD - N))
    src_r = jnp.concatenate([src, pad_idx]).reshape(NW, NBATCH, B)
    dst_r = jnp.concatenate([dst, pad_idx]).reshape(NW, NBATCH, B)

    deg_p = _deg(dst_r)
    g1 = _mm1(x, W1, deg_p)
    acc1 = _scatter(g1, src_r, dst_r)
    g2 = _mm2(acc1, g1, deg_p, b1, W2)
    acc2 = _scatter(g2, src_r, dst_r)
    return _fin(acc2, g2, deg_p, b2)


# speedup vs baseline: 24.1097x; 24.1097x over previous
import jax
import jax.numpy as jnp
from jax import lax
from jax.experimental import pallas as pl
from jax.experimental.pallas import tpu as pltpu
from jax.experimental.pallas import tpu_sc as plsc

N = 10000
E = 320000
D = 128

NC = 2
NS = 16
NW = NC * NS
B = 128
NBATCH = (E + NW * B - 1) // (NW * B)
E_PAD = NW * B * NBATCH
N_PAD = 10240
ROWS_PER_TILE_PAD = N_PAD // NS
ROWS_PER_TILE = N // NS
DEG_W = 16

_mesh = plsc.VectorSubcoreMesh(core_axis_name="c", subcore_axis_name="s")


def _fill_vmem(ref, rows, width, value):
    vv = jnp.full((16,), value, jnp.float32)

    @pl.loop(0, rows)
    def _(i):
        @pl.loop(0, width // 16)
        def _(j):
            ref[i, pl.ds(j * 16, 16)] = vv


def _fill_vmem_1d(ref, n, value):
    vv = jnp.full((16,), value, jnp.float32)

    @pl.loop(0, n // 16)
    def _(i):
        ref[pl.ds(i * 16, 16)] = vv


def _deg_body(dst_hbm, deg_hbm, dstb, zb, onesb, deg_sh, sem):
    c = lax.axis_index("c")
    s = lax.axis_index("s")
    w = s * NC + c

    _fill_vmem_1d(zb, ROWS_PER_TILE_PAD, 0.0)
    pltpu.sync_copy(zb,
                    deg_sh.at[pl.ds(s * ROWS_PER_TILE_PAD, ROWS_PER_TILE_PAD)])
    _fill_vmem_1d(onesb, B, 1.0)

    pltpu.async_copy(dst_hbm.at[w], dstb, sem).wait()
    plsc.subcore_barrier()

    @pl.loop(0, NBATCH)
    def _(j):
        pltpu.sync_copy(onesb, deg_sh.at[dstb.at[j]], add=True)

    plsc.subcore_barrier()
    pltpu.sync_copy(
        deg_sh.at[pl.ds(s * ROWS_PER_TILE_PAD, ROWS_PER_TILE_PAD)],
        deg_hbm.at[pl.ds(c * N_PAD + s * ROWS_PER_TILE_PAD, ROWS_PER_TILE_PAD)])


@jax.jit
def _deg(dst_r):
    return pl.kernel(
        _deg_body,
        out_type=jax.ShapeDtypeStruct((NC * N_PAD,), jnp.float32),
        mesh=_mesh,
        scratch_types=[
            pltpu.VMEM((NBATCH, B), jnp.int32),
            pltpu.VMEM((ROWS_PER_TILE_PAD,), jnp.float32),
            pltpu.VMEM((B,), jnp.float32),
            pltpu.VMEM_SHARED((N_PAD,), jnp.float32),
            pltpu.SemaphoreType.DMA,
        ],
    )(dst_r)


def _scatter_body(g_hbm, src_hbm, dst_hbm, out_hbm, srcb, dstb, rows,
                  acc_sh, sem):
    c = lax.axis_index("c")
    s = lax.axis_index("s")
    w = s * NC + c

    _fill_vmem(rows, B, D, 0.0)
    @pl.loop(0, ROWS_PER_TILE_PAD // B)
    def _(k):
        pltpu.sync_copy(rows, acc_sh.at[pl.ds(s * ROWS_PER_TILE_PAD + k * B, B)])

    pltpu.async_copy(src_hbm.at[w], srcb, sem).wait()
    pltpu.async_copy(dst_hbm.at[w], dstb, sem).wait()
    plsc.subcore_barrier()

    @pl.loop(0, NBATCH)
    def _(j):
        pltpu.async_copy(g_hbm.at[srcb.at[j]], rows, sem).wait()
        pltpu.sync_copy(rows, acc_sh.at[dstb.at[j]], add=True)

    plsc.subcore_barrier()
    pltpu.sync_copy(
        acc_sh.at[pl.ds(s * ROWS_PER_TILE_PAD, ROWS_PER_TILE_PAD)],
        out_hbm.at[pl.ds(c * N_PAD + s * ROWS_PER_TILE_PAD, ROWS_PER_TILE_PAD)])


@jax.jit
def _scatter(g, src_r, dst_r):
    return pl.kernel(
        _scatter_body,
        out_type=jax.ShapeDtypeStruct((NC * N_PAD, D), jnp.float32),
        mesh=_mesh,
        scratch_types=[
            pltpu.VMEM((NBATCH, B), jnp.int32),
            pltpu.VMEM((NBATCH, B), jnp.int32),
            pltpu.VMEM((B, D), jnp.float32),
            pltpu.VMEM_SHARED((N_PAD, D), jnp.float32),
            pltpu.SemaphoreType.DMA,
        ],
    )(g, src_r, dst_r)


def _dis_col(deg_ref):
    deg = (deg_ref[0:N] + deg_ref[N_PAD:N_PAD + N]) + 1.0
    return lax.rsqrt(deg).reshape(N, 1)


def _mm1_body(x_ref, w_ref, deg_ref, g_ref):
    dis = _dis_col(deg_ref)
    h = jnp.dot(x_ref[...], w_ref[...], preferred_element_type=jnp.float32)
    g_ref[0:N, :] = dis * h
    g_ref[N:N_PAD, :] = jnp.zeros((N_PAD - N, D), jnp.float32)


@jax.jit
def _mm1(x, W1, deg_p):
    return pl.pallas_call(
        _mm1_body,
        out_shape=jax.ShapeDtypeStruct((N_PAD, D), jnp.float32),
    )(x, W1, deg_p)


def _mm2_body(acc_ref, g1_ref, deg_ref, b_ref, w_ref, g2_ref):
    dis = _dis_col(deg_ref)
    acc = acc_ref[0:N, :] + acc_ref[N_PAD:N_PAD + N, :]
    h1 = jnp.maximum(dis * (acc + g1_ref[0:N, :]) + b_ref[...], 0.0)
    h2 = jnp.dot(h1, w_ref[...], preferred_element_type=jnp.float32)
    g2_ref[0:N, :] = dis * h2
    g2_ref[N:N_PAD, :] = jnp.zeros((N_PAD - N, D), jnp.float32)


@jax.jit
def _mm2(acc1, g1, deg_p, b1, W2):
    return pl.pallas_call(
        _mm2_body,
        out_shape=jax.ShapeDtypeStruct((N_PAD, D), jnp.float32),
    )(acc1, g1, deg_p, b1, W2)


def _fin_body(acc_ref, g2_ref, deg_ref, b_ref, out_ref):
    dis = _dis_col(deg_ref)
    acc = acc_ref[0:N, :] + acc_ref[N_PAD:N_PAD + N, :]
    out_ref[...] = dis * (acc + g2_ref[0:N, :]) + b_ref[...]


@jax.jit
def _fin(acc2, g2, deg_p, b2):
    return pl.pallas_call(
        _fin_body,
        out_shape=jax.ShapeDtypeStruct((N, D), jnp.float32),
    )(acc2, g2, deg_p, b2)


def kernel(x, edge_index, W1, b1, W2, b2):
    src = edge_index[0].astype(jnp.int32)
    dst = edge_index[1].astype(jnp.int32)
    pad = E_PAD - E
    pad_idx = N + (jnp.arange(pad, dtype=jnp.int32) % (N_PAD - N))
    src_r = jnp.concatenate([src, pad_idx]).reshape(NW, NBATCH, B)
    dst_r = jnp.concatenate([dst, pad_idx]).reshape(NW, NBATCH, B)

    deg_p = _deg(dst_r)
    g1 = _mm1(x, W1, deg_p)
    acc1 = _scatter(g1, src_r, dst_r)
    g2 = _mm2(acc1, g1, deg_p, b1, W2)
    acc2 = _scatter(g2, src_r, dst_r)
    return _fin(acc2, g2, deg_p, b2)


# speedup vs baseline: 33.1360x; 1.3744x over previous
import jax
import jax.numpy as jnp
from jax import lax
from jax.experimental import pallas as pl
from jax.experimental.pallas import tpu as pltpu
from jax.experimental.pallas import tpu_sc as plsc

N = 10000
E = 320000
D = 128

NC = 2
NS = 16
NW = NC * NS
B = 128
SUPER = 16
NSUPER = 5
NBATCH = NSUPER * SUPER
E_PAD = NW * B * NBATCH
N_PAD = 10240
ROWS_PER_TILE_PAD = N_PAD // NS
ROWS_PER_TILE = N // NS
DEG_W = 16

_mesh = plsc.VectorSubcoreMesh(core_axis_name="c", subcore_axis_name="s")


def _fill_vmem(ref, rows, width, value):
    vv = jnp.full((16,), value, jnp.float32)

    @pl.loop(0, rows)
    def _(i):
        @pl.loop(0, width // 16)
        def _(j):
            ref[i, pl.ds(j * 16, 16)] = vv


def _fill_vmem_1d(ref, n, value):
    vv = jnp.full((16,), value, jnp.float32)

    @pl.loop(0, n // 16)
    def _(i):
        ref[pl.ds(i * 16, 16)] = vv


def _deg_body(dst_hbm, deg_hbm, dstb, zb, onesb, deg_sh, sem):
    c = lax.axis_index("c")
    s = lax.axis_index("s")
    w = s * NC + c

    _fill_vmem_1d(zb, ROWS_PER_TILE_PAD, 0.0)
    pltpu.sync_copy(zb,
                    deg_sh.at[pl.ds(s * ROWS_PER_TILE_PAD, ROWS_PER_TILE_PAD)])
    _fill_vmem_1d(onesb, B, 1.0)

    pltpu.async_copy(dst_hbm.at[w], dstb, sem).wait()
    plsc.subcore_barrier()

    @pl.loop(0, NSUPER)
    def _(k):
        @pl.loop(0, SUPER)
        def _(j):
            pltpu.sync_copy(onesb, deg_sh.at[dstb.at[k, j]], add=True)

    plsc.subcore_barrier()
    pltpu.sync_copy(
        deg_sh.at[pl.ds(s * ROWS_PER_TILE_PAD, ROWS_PER_TILE_PAD)],
        deg_hbm.at[pl.ds(c * N_PAD + s * ROWS_PER_TILE_PAD, ROWS_PER_TILE_PAD)])


@jax.jit
def _deg(dst_r):
    return pl.kernel(
        _deg_body,
        out_type=jax.ShapeDtypeStruct((NC * N_PAD,), jnp.float32),
        mesh=_mesh,
        scratch_types=[
            pltpu.VMEM((NSUPER, SUPER, B), jnp.int32),
            pltpu.VMEM((ROWS_PER_TILE_PAD,), jnp.float32),
            pltpu.VMEM((B,), jnp.float32),
            pltpu.VMEM_SHARED((N_PAD,), jnp.float32),
            pltpu.SemaphoreType.DMA,
        ],
    )(dst_r)


def _scatter_body(g_hbm, src_hbm, dst_hbm, out_hbm, srcb, dstb, rows0, rows1,
                  acc_sh, sem0, sem1):
    c = lax.axis_index("c")
    s = lax.axis_index("s")
    w = s * NC + c

    _fill_vmem(rows0, B, D, 0.0)
    @pl.loop(0, ROWS_PER_TILE_PAD // B)
    def _(k):
        pltpu.sync_copy(rows0, acc_sh.at[pl.ds(s * ROWS_PER_TILE_PAD + k * B, B)])
    plsc.subcore_barrier()

    @pl.loop(0, NSUPER)
    def _(k):
        pltpu.sync_copy(src_hbm.at[w, k], srcb)
        pltpu.sync_copy(dst_hbm.at[w, k], dstb)
        pltpu.async_copy(g_hbm.at[srcb.at[0]], rows0, sem0)

        @pl.loop(0, SUPER, step=2)
        def _(i):
            pltpu.async_copy(g_hbm.at[srcb.at[i + 1]], rows1, sem1)
            pltpu.make_async_copy(g_hbm.at[srcb.at[0]], rows0, sem0).wait()
            pltpu.sync_copy(rows0, acc_sh.at[dstb.at[i]], add=True)

            @pl.when(i + 2 < SUPER)
            def _():
                pltpu.async_copy(g_hbm.at[srcb.at[i + 2]], rows0, sem0)

            pltpu.make_async_copy(g_hbm.at[srcb.at[0]], rows1, sem1).wait()
            pltpu.sync_copy(rows1, acc_sh.at[dstb.at[i + 1]], add=True)

    plsc.subcore_barrier()
    pltpu.sync_copy(
        acc_sh.at[pl.ds(s * ROWS_PER_TILE_PAD, ROWS_PER_TILE_PAD)],
        out_hbm.at[pl.ds(c * N_PAD + s * ROWS_PER_TILE_PAD, ROWS_PER_TILE_PAD)])


@jax.jit
def _scatter(g, src_r, dst_r):
    return pl.kernel(
        _scatter_body,
        out_type=jax.ShapeDtypeStruct((NC * N_PAD, D), jnp.float32),
        mesh=_mesh,
        scratch_types=[
            pltpu.VMEM((SUPER, B), jnp.int32),
            pltpu.VMEM((SUPER, B), jnp.int32),
            pltpu.VMEM((B, D), jnp.float32),
            pltpu.VMEM((B, D), jnp.float32),
            pltpu.VMEM_SHARED((N_PAD, D), jnp.float32),
            pltpu.SemaphoreType.DMA,
            pltpu.SemaphoreType.DMA,
        ],
    )(g, src_r, dst_r)


def _dis_col(deg_ref):
    deg = (deg_ref[0:N] + deg_ref[N_PAD:N_PAD + N]) + 1.0
    return lax.rsqrt(deg).reshape(N, 1)


def _mm1_body(x_ref, w_ref, deg_ref, g_ref):
    dis = _dis_col(deg_ref)
    h = jnp.dot(x_ref[...], w_ref[...], preferred_element_type=jnp.float32)
    g_ref[0:N, :] = dis * h
    g_ref[N:N_PAD, :] = jnp.zeros((N_PAD - N, D), jnp.float32)


@jax.jit
def _mm1(x, W1, deg_p):
    return pl.pallas_call(
        _mm1_body,
        out_shape=jax.ShapeDtypeStruct((N_PAD, D), jnp.float32),
    )(x, W1, deg_p)


def _mm2_body(acc_ref, g1_ref, deg_ref, b_ref, w_ref, g2_ref):
    dis = _dis_col(deg_ref)
    acc = acc_ref[0:N, :] + acc_ref[N_PAD:N_PAD + N, :]
    h1 = jnp.maximum(dis * (acc + g1_ref[0:N, :]) + b_ref[...], 0.0)
    h2 = jnp.dot(h1, w_ref[...], preferred_element_type=jnp.float32)
    g2_ref[0:N, :] = dis * h2
    g2_ref[N:N_PAD, :] = jnp.zeros((N_PAD - N, D), jnp.float32)


@jax.jit
def _mm2(acc1, g1, deg_p, b1, W2):
    return pl.pallas_call(
        _mm2_body,
        out_shape=jax.ShapeDtypeStruct((N_PAD, D), jnp.float32),
    )(acc1, g1, deg_p, b1, W2)


def _fin_body(acc_ref, g2_ref, deg_ref, b_ref, out_ref):
    dis = _dis_col(deg_ref)
    acc = acc_ref[0:N, :] + acc_ref[N_PAD:N_PAD + N, :]
    out_ref[...] = dis * (acc + g2_ref[0:N, :]) + b_ref[...]


@jax.jit
def _fin(acc2, g2, deg_p, b2):
    return pl.pallas_call(
        _fin_body,
        out_shape=jax.ShapeDtypeStruct((N, D), jnp.float32),
    )(acc2, g2, deg_p, b2)


def kernel(x, edge_index, W1, b1, W2, b2):
    src = edge_index[0].astype(jnp.int32)
    dst = edge_index[1].astype(jnp.int32)
    pad = E_PAD - E
    pad_idx = N + (jnp.arange(pad, dtype=jnp.int32) % (N_PAD - N))
    src_r = jnp.concatenate([src, pad_idx]).reshape(NW, NSUPER, SUPER, B)
    dst_r = jnp.concatenate([dst, pad_idx]).reshape(NW, NSUPER, SUPER, B)

    deg_p = _deg(dst_r)
    g1 = _mm1(x, W1, deg_p)
    acc1 = _scatter(g1, src_r, dst_r)
    g2 = _mm2(acc1, g1, deg_p, b1, W2)
    acc2 = _scatter(g2, src_r, dst_r)
    return _fin(acc2, g2, deg_p, b2)


# speedup vs baseline: 35.5962x; 1.0742x over previous
import jax
import jax.numpy as jnp
from jax import lax
from jax.experimental import pallas as pl
from jax.experimental.pallas import tpu as pltpu
from jax.experimental.pallas import tpu_sc as plsc

N = 10000
E = 320000
D = 128

NC = 2
NS = 16
NW = NC * NS
B = 128
SUPER = 20
NSUPER = 4
NBATCH = NSUPER * SUPER
E_PAD = NW * B * NBATCH
N_PAD = 10240
ROWS_PER_TILE_PAD = N_PAD // NS
ROWS_PER_TILE = N // NS
DEG_W = 16

_mesh = plsc.VectorSubcoreMesh(core_axis_name="c", subcore_axis_name="s")


def _fill_vmem(ref, rows, width, value):
    vv = jnp.full((16,), value, jnp.float32)

    @pl.loop(0, rows)
    def _(i):
        @pl.loop(0, width // 16)
        def _(j):
            ref[i, pl.ds(j * 16, 16)] = vv


def _fill_vmem_1d(ref, n, value):
    vv = jnp.full((16,), value, jnp.float32)

    @pl.loop(0, n // 16)
    def _(i):
        ref[pl.ds(i * 16, 16)] = vv


def _deg_body(dst_hbm, deg_hbm, dstb, zb, onesb, deg_sh, sem):
    c = lax.axis_index("c")
    s = lax.axis_index("s")
    w = s * NC + c

    _fill_vmem_1d(zb, ROWS_PER_TILE_PAD, 0.0)
    pltpu.sync_copy(zb,
                    deg_sh.at[pl.ds(s * ROWS_PER_TILE_PAD, ROWS_PER_TILE_PAD)])
    _fill_vmem_1d(onesb, B, 1.0)

    pltpu.async_copy(dst_hbm.at[w], dstb, sem).wait()
    plsc.subcore_barrier()

    @pl.loop(0, NSUPER)
    def _(k):
        @pl.loop(0, SUPER)
        def _(j):
            pltpu.sync_copy(onesb, deg_sh.at[dstb.at[k, j]], add=True)

    plsc.subcore_barrier()
    pltpu.sync_copy(
        deg_sh.at[pl.ds(s * ROWS_PER_TILE_PAD, ROWS_PER_TILE_PAD)],
        deg_hbm.at[pl.ds(c * N_PAD + s * ROWS_PER_TILE_PAD, ROWS_PER_TILE_PAD)])


@jax.jit
def _deg(dst_r):
    return pl.kernel(
        _deg_body,
        out_type=jax.ShapeDtypeStruct((NC * N_PAD,), jnp.float32),
        mesh=_mesh,
        scratch_types=[
            pltpu.VMEM((NSUPER, SUPER, B), jnp.int32),
            pltpu.VMEM((ROWS_PER_TILE_PAD,), jnp.float32),
            pltpu.VMEM((B,), jnp.float32),
            pltpu.VMEM_SHARED((N_PAD,), jnp.float32),
            pltpu.SemaphoreType.DMA,
        ],
    )(dst_r)


def _scatter_body(g_hbm, src_hbm, dst_hbm, out_hbm, srcb, dstb, rows0, rows1,
                  acc_sh, sem0, sem1, semi):
    c = lax.axis_index("c")
    s = lax.axis_index("s")
    w = s * NC + c

    _fill_vmem(rows0, B, D, 0.0)
    @pl.loop(0, ROWS_PER_TILE_PAD // B)
    def _(k):
        pltpu.sync_copy(rows0, acc_sh.at[pl.ds(s * ROWS_PER_TILE_PAD + k * B, B)])
    plsc.subcore_barrier()

    def _gather(t, rows, sem):
        kt = t // SUPER
        pltpu.async_copy(g_hbm.at[srcb.at[kt % 2, t % SUPER]], rows, sem)

    def _scat(t, rows):
        kt = t // SUPER
        pltpu.sync_copy(rows, acc_sh.at[dstb.at[kt % 2, t % SUPER]], add=True)

    pltpu.sync_copy(src_hbm.at[w, 0], srcb.at[0])
    pltpu.sync_copy(dst_hbm.at[w, 0], dstb.at[0])
    _gather(0, rows0, sem0)

    @pl.loop(0, NBATCH, step=2)
    def _(j):
        k = j // SUPER
        i = j % SUPER

        _gather(j + 1, rows1, sem1)
        pltpu.make_async_copy(g_hbm.at[srcb.at[0, 0]], rows0, sem0).wait()
        _scat(j, rows0)

        @pl.when((i == 2) & (k + 1 < NSUPER))
        def _():
            pltpu.async_copy(src_hbm.at[w, k + 1], srcb.at[(k + 1) % 2], semi)
            pltpu.async_copy(dst_hbm.at[w, k + 1], dstb.at[(k + 1) % 2], semi)

        @pl.when(j + 2 < NBATCH)
        def _():
            @pl.when((j + 2) % SUPER == 0)
            def _():
                pltpu.make_async_copy(src_hbm.at[w, 0], srcb.at[0], semi).wait()
                pltpu.make_async_copy(dst_hbm.at[w, 0], dstb.at[0], semi).wait()
            _gather(j + 2, rows0, sem0)

        pltpu.make_async_copy(g_hbm.at[srcb.at[0, 0]], rows1, sem1).wait()
        _scat(j + 1, rows1)

    plsc.subcore_barrier()
    pltpu.sync_copy(
        acc_sh.at[pl.ds(s * ROWS_PER_TILE_PAD, ROWS_PER_TILE_PAD)],
        out_hbm.at[pl.ds(c * N_PAD + s * ROWS_PER_TILE_PAD, ROWS_PER_TILE_PAD)])


@jax.jit
def _scatter(g, src_r, dst_r):
    return pl.kernel(
        _scatter_body,
        out_type=jax.ShapeDtypeStruct((NC * N_PAD, D), jnp.float32),
        mesh=_mesh,
        scratch_types=[
            pltpu.VMEM((2, SUPER, B), jnp.int32),
            pltpu.VMEM((2, SUPER, B), jnp.int32),
            pltpu.VMEM((B, D), jnp.float32),
            pltpu.VMEM((B, D), jnp.float32),
            pltpu.VMEM_SHARED((N_PAD, D), jnp.float32),
            pltpu.SemaphoreType.DMA,
            pltpu.SemaphoreType.DMA,
            pltpu.SemaphoreType.DMA,
        ],
    )(g, src_r, dst_r)


def _dis_col(deg_ref):
    deg = (deg_ref[0:N] + deg_ref[N_PAD:N_PAD + N]) + 1.0
    return lax.rsqrt(deg).reshape(N, 1)


def _mm1_body(x_ref, w_ref, deg_ref, g_ref):
    dis = _dis_col(deg_ref)
    h = jnp.dot(x_ref[...], w_ref[...], preferred_element_type=jnp.float32)
    g_ref[0:N, :] = dis * h
    g_ref[N:N_PAD, :] = jnp.zeros((N_PAD - N, D), jnp.float32)


@jax.jit
def _mm1(x, W1, deg_p):
    return pl.pallas_call(
        _mm1_body,
        out_shape=jax.ShapeDtypeStruct((N_PAD, D), jnp.float32),
    )(x, W1, deg_p)


def _mm2_body(acc_ref, g1_ref, deg_ref, b_ref, w_ref, g2_ref):
    dis = _dis_col(deg_ref)
    acc = acc_ref[0:N, :] + acc_ref[N_PAD:N_PAD + N, :]
    h1 = jnp.maximum(dis * (acc + g1_ref[0:N, :]) + b_ref[...], 0.0)
    h2 = jnp.dot(h1, w_ref[...], preferred_element_type=jnp.float32)
    g2_ref[0:N, :] = dis * h2
    g2_ref[N:N_PAD, :] = jnp.zeros((N_PAD - N, D), jnp.float32)


@jax.jit
def _mm2(acc1, g1, deg_p, b1, W2):
    return pl.pallas_call(
        _mm2_body,
        out_shape=jax.ShapeDtypeStruct((N_PAD, D), jnp.float32),
    )(acc1, g1, deg_p, b1, W2)


def _fin_body(acc_ref, g2_ref, deg_ref, b_ref, out_ref):
    dis = _dis_col(deg_ref)
    acc = acc_ref[0:N, :] + acc_ref[N_PAD:N_PAD + N, :]
    out_ref[...] = dis * (acc + g2_ref[0:N, :]) + b_ref[...]


@jax.jit
def _fin(acc2, g2, deg_p, b2):
    return pl.pallas_call(
        _fin_body,
        out_shape=jax.ShapeDtypeStruct((N, D), jnp.float32),
    )(acc2, g2, deg_p, b2)


def kernel(x, edge_index, W1, b1, W2, b2):
    src = edge_index[0].astype(jnp.int32)
    dst = edge_index[1].astype(jnp.int32)
    pad = E_PAD - E
    pad_idx = N + (jnp.arange(pad, dtype=jnp.int32) % (N_PAD - N))
    src_r = jnp.concatenate([src, pad_idx]).reshape(NW, NSUPER, SUPER, B)
    dst_r = jnp.concatenate([dst, pad_idx]).reshape(NW, NSUPER, SUPER, B)

    deg_p = _deg(dst_r)
    g1 = _mm1(x, W1, deg_p)
    acc1 = _scatter(g1, src_r, dst_r)
    g2 = _mm2(acc1, g1, deg_p, b1, W2)
    acc2 = _scatter(g2, src_r, dst_r)
    return _fin(acc2, g2, deg_p, b2)


# speedup vs baseline: 36.1945x; 1.0168x over previous
import jax
import jax.numpy as jnp
from jax import lax
from jax.experimental import pallas as pl
from jax.experimental.pallas import tpu as pltpu
from jax.experimental.pallas import tpu_sc as plsc

N = 10000
E = 320000
D = 128

NC = 2
NS = 16
NW = NC * NS
B = 128
SUPER = 16
NSUPER = 5
NBATCH = NSUPER * SUPER
E_PAD = NW * B * NBATCH
N_PAD = 10240
ROWS_PER_TILE_PAD = N_PAD // NS
ROWS_PER_TILE = N // NS
DEG_W = 16

_mesh = plsc.VectorSubcoreMesh(core_axis_name="c", subcore_axis_name="s")


def _fill_vmem(ref, rows, width, value):
    vv = jnp.full((16,), value, jnp.float32)

    @pl.loop(0, rows)
    def _(i):
        @pl.loop(0, width // 16)
        def _(j):
            ref[i, pl.ds(j * 16, 16)] = vv


def _fill_vmem_1d(ref, n, value):
    vv = jnp.full((16,), value, jnp.float32)

    @pl.loop(0, n // 16)
    def _(i):
        ref[pl.ds(i * 16, 16)] = vv


def _deg_body(er_hbm, deg_hbm, dstb, zb, onesb, deg_sh, sem):
    c = lax.axis_index("c")
    s = lax.axis_index("s")
    w = s * NC + c

    _fill_vmem_1d(zb, ROWS_PER_TILE_PAD, 0.0)
    pltpu.sync_copy(zb,
                    deg_sh.at[pl.ds(s * ROWS_PER_TILE_PAD, ROWS_PER_TILE_PAD)])
    _fill_vmem_1d(onesb, B, 1.0)

    pltpu.async_copy(er_hbm.at[1, w], dstb, sem).wait()
    plsc.subcore_barrier()

    @pl.loop(0, NSUPER)
    def _(k):
        @pl.loop(0, SUPER)
        def _(j):
            pltpu.sync_copy(onesb, deg_sh.at[dstb.at[k, j]], add=True)

    plsc.subcore_barrier()
    pltpu.sync_copy(
        deg_sh.at[pl.ds(s * ROWS_PER_TILE_PAD, ROWS_PER_TILE_PAD)],
        deg_hbm.at[pl.ds(c * N_PAD + s * ROWS_PER_TILE_PAD, ROWS_PER_TILE_PAD)])


@jax.jit
def _deg(er):
    return pl.kernel(
        _deg_body,
        out_type=jax.ShapeDtypeStruct((NC * N_PAD,), jnp.float32),
        mesh=_mesh,
        scratch_types=[
            pltpu.VMEM((NSUPER, SUPER, B), jnp.int32),
            pltpu.VMEM((ROWS_PER_TILE_PAD,), jnp.float32),
            pltpu.VMEM((B,), jnp.float32),
            pltpu.VMEM_SHARED((N_PAD,), jnp.float32),
            pltpu.SemaphoreType.DMA,
        ],
    )(er)


def _scatter_body(g_hbm, er_hbm, out_hbm, srcb, dstb, rows0, rows1,
                  acc_sh, sem0, sem1, semi):
    c = lax.axis_index("c")
    s = lax.axis_index("s")
    w = s * NC + c

    _fill_vmem(rows0, B, D, 0.0)
    @pl.loop(0, ROWS_PER_TILE_PAD // B)
    def _(k):
        pltpu.sync_copy(rows0, acc_sh.at[pl.ds(s * ROWS_PER_TILE_PAD + k * B, B)])
    plsc.subcore_barrier()

    def _gather(t, rows, sem):
        kt = t // SUPER
        pltpu.async_copy(g_hbm.at[srcb.at[kt % 2, t % SUPER]], rows, sem)

    def _scat(t, rows):
        kt = t // SUPER
        pltpu.sync_copy(rows, acc_sh.at[dstb.at[kt % 2, t % SUPER]], add=True)

    pltpu.sync_copy(er_hbm.at[0, w, 0], srcb.at[0])
    pltpu.sync_copy(er_hbm.at[1, w, 0], dstb.at[0])
    _gather(0, rows0, sem0)

    @pl.loop(0, NBATCH, step=2)
    def _(j):
        k = j // SUPER
        i = j % SUPER

        _gather(j + 1, rows1, sem1)
        pltpu.make_async_copy(g_hbm.at[srcb.at[0, 0]], rows0, sem0).wait()
        _scat(j, rows0)

        @pl.when((i == 2) & (k + 1 < NSUPER))
        def _():
            pltpu.async_copy(er_hbm.at[0, w, k + 1], srcb.at[(k + 1) % 2], semi)
            pltpu.async_copy(er_hbm.at[1, w, k + 1], dstb.at[(k + 1) % 2], semi)

        @pl.when(j + 2 < NBATCH)
        def _():
            @pl.when((j + 2) % SUPER == 0)
            def _():
                pltpu.make_async_copy(er_hbm.at[0, w, 0], srcb.at[0], semi).wait()
                pltpu.make_async_copy(er_hbm.at[1, w, 0], dstb.at[0], semi).wait()
            _gather(j + 2, rows0, sem0)

        pltpu.make_async_copy(g_hbm.at[srcb.at[0, 0]], rows1, sem1).wait()
        _scat(j + 1, rows1)

    plsc.subcore_barrier()
    pltpu.sync_copy(
        acc_sh.at[pl.ds(s * ROWS_PER_TILE_PAD, ROWS_PER_TILE_PAD)],
        out_hbm.at[pl.ds(c * N_PAD + s * ROWS_PER_TILE_PAD, ROWS_PER_TILE_PAD)])


@jax.jit
def _scatter(g, er):
    return pl.kernel(
        _scatter_body,
        out_type=jax.ShapeDtypeStruct((NC * N_PAD, D), jnp.float32),
        mesh=_mesh,
        scratch_types=[
            pltpu.VMEM((2, SUPER, B), jnp.int32),
            pltpu.VMEM((2, SUPER, B), jnp.int32),
            pltpu.VMEM((B, D), jnp.float32),
            pltpu.VMEM((B, D), jnp.float32),
            pltpu.VMEM_SHARED((N_PAD, D), jnp.float32),
            pltpu.SemaphoreType.DMA,
            pltpu.SemaphoreType.DMA,
            pltpu.SemaphoreType.DMA,
        ],
    )(g, er)


def _dis_col(deg_ref):
    deg = (deg_ref[0:N] + deg_ref[N_PAD:N_PAD + N]) + 1.0
    return lax.rsqrt(deg).reshape(N, 1)


def _mmx_body(x_ref, w_ref, u_ref):
    u_ref[...] = jnp.dot(x_ref[...], w_ref[...],
                         preferred_element_type=jnp.float32)


@jax.jit
def _mmx(x, W1):
    return pl.pallas_call(
        _mmx_body,
        out_shape=jax.ShapeDtypeStruct((N, D), jnp.float32),
    )(x, W1)


def _scale_body(u_ref, deg_ref, g_ref):
    dis = _dis_col(deg_ref)
    g_ref[0:N, :] = dis * u_ref[...]
    g_ref[N:N_PAD, :] = jnp.zeros((N_PAD - N, D), jnp.float32)


@jax.jit
def _scale1(u1, deg_p):
    return pl.pallas_call(
        _scale_body,
        out_shape=jax.ShapeDtypeStruct((N_PAD, D), jnp.float32),
    )(u1, deg_p)


def _lin_body(g_ref, deg_ref, b_ref, o_ref):
    dis = _dis_col(deg_ref)
    o_ref[...] = dis * g_ref[0:N, :] + b_ref[...]


@jax.jit
def _lin(g, deg_p, b):
    return pl.pallas_call(
        _lin_body,
        out_shape=jax.ShapeDtypeStruct((N, D), jnp.float32),
    )(g, deg_p, b)


def _mm2_body(acc_ref, lin_ref, deg_ref, w_ref, g2_ref):
    dis = _dis_col(deg_ref)
    acc = acc_ref[0:N, :] + acc_ref[N_PAD:N_PAD + N, :]
    h1 = jnp.maximum(dis * acc + lin_ref[...], 0.0)
    h2 = jnp.dot(h1, w_ref[...], preferred_element_type=jnp.float32)
    g2_ref[0:N, :] = dis * h2
    g2_ref[N:N_PAD, :] = jnp.zeros((N_PAD - N, D), jnp.float32)


@jax.jit
def _mm2(acc1, lin1, deg_p, W2):
    return pl.pallas_call(
        _mm2_body,
        out_shape=jax.ShapeDtypeStruct((N_PAD, D), jnp.float32),
    )(acc1, lin1, deg_p, W2)


def _fin_body(acc_ref, lin_ref, deg_ref, out_ref):
    dis = _dis_col(deg_ref)
    acc = acc_ref[0:N, :] + acc_ref[N_PAD:N_PAD + N, :]
    out_ref[...] = dis * acc + lin_ref[...]


@jax.jit
def _fin(acc2, lin2, deg_p):
    return pl.pallas_call(
        _fin_body,
        out_shape=jax.ShapeDtypeStruct((N, D), jnp.float32),
    )(acc2, lin2, deg_p)


def kernel(x, edge_index, W1, b1, W2, b2):
    pad = E_PAD - E
    pad_idx = N + (jnp.arange(pad, dtype=jnp.int32) % (N_PAD - N))
    er = jnp.concatenate(
        [edge_index.astype(jnp.int32),
         jnp.broadcast_to(pad_idx, (2, pad))], axis=1,
    ).reshape(2, NW, NSUPER, SUPER, B)

    deg_p = _deg(er)
    u1 = _mmx(x, W1)
    g1 = _scale1(u1, deg_p)
    acc1 = _scatter(g1, er)
    lin1 = _lin(g1, deg_p, b1)
    g2 = _mm2(acc1, lin1, deg_p, W2)
    acc2 = _scatter(g2, er)
    lin2 = _lin(g2, deg_p, b2)
    return _fin(acc2, lin2, deg_p)


# speedup vs baseline: 36.4831x; 1.0080x over previous
import jax
import jax.numpy as jnp
from jax import lax
from jax.experimental import pallas as pl
from jax.experimental.pallas import tpu as pltpu
from jax.experimental.pallas import tpu_sc as plsc

N = 10000
E = 320000
D = 128

NC = 2
NS = 16
NW = NC * NS
B = 128
SUPER = 16
NSUPER = 5
NBATCH = NSUPER * SUPER
E_PAD = NW * B * NBATCH
N_PAD = 10240
ROWS_PER_TILE_PAD = N_PAD // NS
ROWS_PER_TILE = N // NS
DEG_W = 16

_mesh = plsc.VectorSubcoreMesh(core_axis_name="c", subcore_axis_name="s")


def _fill_vmem(ref, rows, width, value):
    vv = jnp.full((16,), value, jnp.float32)

    @pl.loop(0, rows)
    def _(i):
        @pl.loop(0, width // 16)
        def _(j):
            ref[i, pl.ds(j * 16, 16)] = vv


def _fill_vmem_1d(ref, n, value):
    vv = jnp.full((16,), value, jnp.float32)

    @pl.loop(0, n // 16)
    def _(i):
        ref[pl.ds(i * 16, 16)] = vv


def _deg_body(er_hbm, deg_hbm, dstb, zb, onesb, deg_sh, sem):
    c = lax.axis_index("c")
    s = lax.axis_index("s")
    w = s * NC + c

    _fill_vmem_1d(zb, ROWS_PER_TILE_PAD, 0.0)
    pltpu.sync_copy(zb,
                    deg_sh.at[pl.ds(s * ROWS_PER_TILE_PAD, ROWS_PER_TILE_PAD)])
    _fill_vmem_1d(onesb, B, 1.0)

    pltpu.async_copy(er_hbm.at[1, pl.ds(w * NBATCH, NBATCH)], dstb, sem).wait()
    plsc.subcore_barrier()

    @pl.loop(0, NBATCH)
    def _(j):
        pltpu.sync_copy(onesb, deg_sh.at[dstb.at[j]], add=True)

    plsc.subcore_barrier()
    pltpu.sync_copy(
        deg_sh.at[pl.ds(s * ROWS_PER_TILE_PAD, ROWS_PER_TILE_PAD)],
        deg_hbm.at[pl.ds(c * N_PAD + s * ROWS_PER_TILE_PAD, ROWS_PER_TILE_PAD)])


@jax.jit
def _deg(er):
    return pl.kernel(
        _deg_body,
        out_type=jax.ShapeDtypeStruct((NC * N_PAD,), jnp.float32),
        mesh=_mesh,
        scratch_types=[
            pltpu.VMEM((NBATCH, B), jnp.int32),
            pltpu.VMEM((ROWS_PER_TILE_PAD,), jnp.float32),
            pltpu.VMEM((B,), jnp.float32),
            pltpu.VMEM_SHARED((N_PAD,), jnp.float32),
            pltpu.SemaphoreType.DMA,
        ],
    )(er)


def _scatter_body(g_hbm, er_hbm, out_hbm, srcb, dstb, rows0, rows1,
                  acc_sh, sem0, sem1, semi):
    c = lax.axis_index("c")
    s = lax.axis_index("s")
    w = s * NC + c

    _fill_vmem(rows0, B, D, 0.0)
    @pl.loop(0, ROWS_PER_TILE_PAD // B)
    def _(k):
        pltpu.sync_copy(rows0, acc_sh.at[pl.ds(s * ROWS_PER_TILE_PAD + k * B, B)])
    plsc.subcore_barrier()

    def _gather(t, rows, sem):
        kt = t // SUPER
        pltpu.async_copy(g_hbm.at[srcb.at[kt % 2, t % SUPER]], rows, sem)

    def _scat(t, rows):
        kt = t // SUPER
        pltpu.sync_copy(rows, acc_sh.at[dstb.at[kt % 2, t % SUPER]], add=True)

    def _idx_src(k):
        return er_hbm.at[0, pl.ds(w * NBATCH + k * SUPER, SUPER)]

    def _idx_dst(k):
        return er_hbm.at[1, pl.ds(w * NBATCH + k * SUPER, SUPER)]

    pltpu.sync_copy(_idx_src(0), srcb.at[0])
    pltpu.sync_copy(_idx_dst(0), dstb.at[0])
    _gather(0, rows0, sem0)

    @pl.loop(0, NBATCH, step=2)
    def _(j):
        k = j // SUPER
        i = j % SUPER

        _gather(j + 1, rows1, sem1)
        pltpu.make_async_copy(g_hbm.at[srcb.at[0, 0]], rows0, sem0).wait()
        _scat(j, rows0)

        @pl.when((i == 2) & (k + 1 < NSUPER))
        def _():
            pltpu.async_copy(_idx_src(k + 1), srcb.at[(k + 1) % 2], semi)
            pltpu.async_copy(_idx_dst(k + 1), dstb.at[(k + 1) % 2], semi)

        @pl.when(j + 2 < NBATCH)
        def _():
            @pl.when((j + 2) % SUPER == 0)
            def _():
                pltpu.make_async_copy(_idx_src(0), srcb.at[0], semi).wait()
                pltpu.make_async_copy(_idx_dst(0), dstb.at[0], semi).wait()
            _gather(j + 2, rows0, sem0)

        pltpu.make_async_copy(g_hbm.at[srcb.at[0, 0]], rows1, sem1).wait()
        _scat(j + 1, rows1)

    plsc.subcore_barrier()
    pltpu.sync_copy(
        acc_sh.at[pl.ds(s * ROWS_PER_TILE_PAD, ROWS_PER_TILE_PAD)],
        out_hbm.at[pl.ds(c * N_PAD + s * ROWS_PER_TILE_PAD, ROWS_PER_TILE_PAD)])


@jax.jit
def _scatter(g, er):
    return pl.kernel(
        _scatter_body,
        out_type=jax.ShapeDtypeStruct((NC * N_PAD, D), jnp.float32),
        mesh=_mesh,
        scratch_types=[
            pltpu.VMEM((2, SUPER, B), jnp.int32),
            pltpu.VMEM((2, SUPER, B), jnp.int32),
            pltpu.VMEM((B, D), jnp.float32),
            pltpu.VMEM((B, D), jnp.float32),
            pltpu.VMEM_SHARED((N_PAD, D), jnp.float32),
            pltpu.SemaphoreType.DMA,
            pltpu.SemaphoreType.DMA,
            pltpu.SemaphoreType.DMA,
        ],
    )(g, er)


def _dis_col(deg_ref):
    deg = (deg_ref[0:N] + deg_ref[N_PAD:N_PAD + N]) + 1.0
    return lax.rsqrt(deg).reshape(N, 1)


def _pad_body(ei_ref, er_ref):
    er_ref[:, 0:E // B, :] = ei_ref[...]
    flat = jax.lax.broadcasted_iota(jnp.int32, (E_PAD // B - E // B, B), 0) * B \
        + jax.lax.broadcasted_iota(jnp.int32, (E_PAD // B - E // B, B), 1)
    padv = N + (flat % (N_PAD - N))
    er_ref[0, E // B:E_PAD // B, :] = padv
    er_ref[1, E // B:E_PAD // B, :] = padv


@jax.jit
def _pad(ei):
    return pl.pallas_call(
        _pad_body,
        out_shape=jax.ShapeDtypeStruct((2, E_PAD // B, B), jnp.int32),
    )(ei)


def _mmx_body(x_ref, w_ref, u_ref):
    u_ref[...] = jnp.dot(x_ref[...], w_ref[...],
                         preferred_element_type=jnp.float32)


@jax.jit
def _mmx(x, W1):
    return pl.pallas_call(
        _mmx_body,
        out_shape=jax.ShapeDtypeStruct((N, D), jnp.float32),
    )(x, W1)


def _scale_body(u_ref, deg_ref, g_ref):
    dis = _dis_col(deg_ref)
    g_ref[0:N, :] = dis * u_ref[...]
    g_ref[N:N_PAD, :] = jnp.zeros((N_PAD - N, D), jnp.float32)


@jax.jit
def _scale1(u1, deg_p):
    return pl.pallas_call(
        _scale_body,
        out_shape=jax.ShapeDtypeStruct((N_PAD, D), jnp.float32),
    )(u1, deg_p)


def _lin_body(g_ref, deg_ref, b_ref, o_ref):
    dis = _dis_col(deg_ref)
    o_ref[...] = dis * g_ref[0:N, :] + b_ref[...]


@jax.jit
def _lin(g, deg_p, b):
    return pl.pallas_call(
        _lin_body,
        out_shape=jax.ShapeDtypeStruct((N, D), jnp.float32),
    )(g, deg_p, b)


def _mm2_body(acc_ref, lin_ref, deg_ref, w_ref, g2_ref):
    dis = _dis_col(deg_ref)
    acc = acc_ref[0:N, :] + acc_ref[N_PAD:N_PAD + N, :]
    h1 = jnp.maximum(dis * acc + lin_ref[...], 0.0)
    h2 = jnp.dot(h1, w_ref[...], preferred_element_type=jnp.float32)
    g2_ref[0:N, :] = dis * h2
    g2_ref[N:N_PAD, :] = jnp.zeros((N_PAD - N, D), jnp.float32)


@jax.jit
def _mm2(acc1, lin1, deg_p, W2):
    return pl.pallas_call(
        _mm2_body,
        out_shape=jax.ShapeDtypeStruct((N_PAD, D), jnp.float32),
    )(acc1, lin1, deg_p, W2)


def _fin_body(acc_ref, lin_ref, deg_ref, out_ref):
    dis = _dis_col(deg_ref)
    acc = acc_ref[0:N, :] + acc_ref[N_PAD:N_PAD + N, :]
    out_ref[...] = dis * acc + lin_ref[...]


@jax.jit
def _fin(acc2, lin2, deg_p):
    return pl.pallas_call(
        _fin_body,
        out_shape=jax.ShapeDtypeStruct((N, D), jnp.float32),
    )(acc2, lin2, deg_p)


def kernel(x, edge_index, W1, b1, W2, b2):
    er = _pad(edge_index.astype(jnp.int32).reshape(2, E // B, B))

    deg_p = _deg(er)
    u1 = _mmx(x, W1)
    g1 = _scale1(u1, deg_p)
    acc1 = _scatter(g1, er)
    lin1 = _lin(g1, deg_p, b1)
    g2 = _mm2(acc1, lin1, deg_p, W2)
    acc2 = _scatter(g2, er)
    lin2 = _lin(g2, deg_p, b2)
    return _fin(acc2, lin2, deg_p)


# speedup vs baseline: 36.9739x; 1.0135x over previous
import jax
import jax.numpy as jnp
from jax import lax
from jax.experimental import pallas as pl
from jax.experimental.pallas import tpu as pltpu
from jax.experimental.pallas import tpu_sc as plsc

N = 10000
E = 320000
D = 128

NC = 2
NS = 16
NW = NC * NS
B = 128
SUPER = 16
NSUPER = 5
NBATCH = NSUPER * SUPER
E_PAD = NW * B * NBATCH
N_PAD = 10240
ROWS_PER_TILE_PAD = N_PAD // NS
ROWS_PER_TILE = N // NS
DEG_W = 16

_mesh = plsc.VectorSubcoreMesh(core_axis_name="c", subcore_axis_name="s")


def _fill_vmem(ref, rows, width, value):
    vv = jnp.full((16,), value, jnp.float32)

    @pl.loop(0, rows)
    def _(i):
        @pl.loop(0, width // 16)
        def _(j):
            ref[i, pl.ds(j * 16, 16)] = vv


def _fill_vmem_1d(ref, n, value):
    vv = jnp.full((16,), value, jnp.float32)

    @pl.loop(0, n // 16)
    def _(i):
        ref[pl.ds(i * 16, 16)] = vv


def _deg_body(er_hbm, deg_hbm, dstb, zb, onesb, deg_sh, sem):
    c = lax.axis_index("c")
    s = lax.axis_index("s")
    w = s * NC + c

    _fill_vmem_1d(zb, ROWS_PER_TILE_PAD, 0.0)
    pltpu.sync_copy(zb,
                    deg_sh.at[pl.ds(s * ROWS_PER_TILE_PAD, ROWS_PER_TILE_PAD)])
    _fill_vmem_1d(onesb, B, 1.0)

    pltpu.async_copy(er_hbm.at[1, pl.ds(w * NBATCH, NBATCH)], dstb, sem).wait()
    plsc.subcore_barrier()

    @pl.loop(0, NBATCH)
    def _(j):
        pltpu.sync_copy(onesb, deg_sh.at[dstb.at[j]], add=True)

    plsc.subcore_barrier()
    pltpu.sync_copy(
        deg_sh.at[pl.ds(s * ROWS_PER_TILE_PAD, ROWS_PER_TILE_PAD)],
        deg_hbm.at[pl.ds(c * N_PAD + s * ROWS_PER_TILE_PAD, ROWS_PER_TILE_PAD)])


@jax.jit
def _deg(er):
    return pl.kernel(
        _deg_body,
        out_type=jax.ShapeDtypeStruct((NC * N_PAD,), jnp.float32),
        mesh=_mesh,
        scratch_types=[
            pltpu.VMEM((NBATCH, B), jnp.int32),
            pltpu.VMEM((ROWS_PER_TILE_PAD,), jnp.float32),
            pltpu.VMEM((B,), jnp.float32),
            pltpu.VMEM_SHARED((N_PAD,), jnp.float32),
            pltpu.SemaphoreType.DMA,
        ],
    )(er)


def _scatter_body(g_hbm, er_hbm, out_hbm, srcb, dstb, rows0, rows1,
                  acc_sh, sem0, sem1, semi):
    c = lax.axis_index("c")
    s = lax.axis_index("s")
    w = s * NC + c

    _fill_vmem(rows0, B, D, 0.0)
    @pl.loop(0, ROWS_PER_TILE_PAD // B)
    def _(k):
        pltpu.sync_copy(rows0, acc_sh.at[pl.ds(s * ROWS_PER_TILE_PAD + k * B, B)])
    plsc.subcore_barrier()

    def _gather(t, rows, sem):
        kt = t // SUPER
        pltpu.async_copy(g_hbm.at[srcb.at[kt % 2, t % SUPER]], rows, sem)

    def _scat(t, rows):
        kt = t // SUPER
        pltpu.sync_copy(rows, acc_sh.at[dstb.at[kt % 2, t % SUPER]], add=True)

    def _idx_src(k):
        return er_hbm.at[0, pl.ds(w * NBATCH + k * SUPER, SUPER)]

    def _idx_dst(k):
        return er_hbm.at[1, pl.ds(w * NBATCH + k * SUPER, SUPER)]

    pltpu.sync_copy(_idx_src(0), srcb.at[0])
    pltpu.sync_copy(_idx_dst(0), dstb.at[0])
    _gather(0, rows0, sem0)

    @pl.loop(0, NBATCH, step=2)
    def _(j):
        k = j // SUPER
        i = j % SUPER

        _gather(j + 1, rows1, sem1)
        pltpu.make_async_copy(g_hbm.at[srcb.at[0, 0]], rows0, sem0).wait()
        _scat(j, rows0)

        @pl.when((i == 2) & (k + 1 < NSUPER))
        def _():
            pltpu.async_copy(_idx_src(k + 1), srcb.at[(k + 1) % 2], semi)
            pltpu.async_copy(_idx_dst(k + 1), dstb.at[(k + 1) % 2], semi)

        @pl.when(j + 2 < NBATCH)
        def _():
            @pl.when((j + 2) % SUPER == 0)
            def _():
                pltpu.make_async_copy(_idx_src(0), srcb.at[0], semi).wait()
                pltpu.make_async_copy(_idx_dst(0), dstb.at[0], semi).wait()
            _gather(j + 2, rows0, sem0)

        pltpu.make_async_copy(g_hbm.at[srcb.at[0, 0]], rows1, sem1).wait()
        _scat(j + 1, rows1)

    plsc.subcore_barrier()
    pltpu.sync_copy(
        acc_sh.at[pl.ds(s * ROWS_PER_TILE_PAD, ROWS_PER_TILE_PAD)],
        out_hbm.at[pl.ds(c * N_PAD + s * ROWS_PER_TILE_PAD, ROWS_PER_TILE_PAD)])


@jax.jit
def _scatter(g, er):
    return pl.kernel(
        _scatter_body,
        out_type=jax.ShapeDtypeStruct((NC * N_PAD, D), jnp.float32),
        mesh=_mesh,
        scratch_types=[
            pltpu.VMEM((2, SUPER, B), jnp.int32),
            pltpu.VMEM((2, SUPER, B), jnp.int32),
            pltpu.VMEM((B, D), jnp.float32),
            pltpu.VMEM((B, D), jnp.float32),
            pltpu.VMEM_SHARED((N_PAD, D), jnp.float32),
            pltpu.SemaphoreType.DMA,
            pltpu.SemaphoreType.DMA,
            pltpu.SemaphoreType.DMA,
        ],
    )(g, er)


def _dis_col(deg_ref):
    deg = (deg_ref[0:N] + deg_ref[N_PAD:N_PAD + N]) + 1.0
    return lax.rsqrt(deg).reshape(N, 1)


def _pad_body(ei_ref, er_ref):
    er_ref[:, 0:E // B, :] = ei_ref[...].reshape(2, E // B, B)
    flat = jax.lax.broadcasted_iota(jnp.int32, (E_PAD // B - E // B, B), 0) * B \
        + jax.lax.broadcasted_iota(jnp.int32, (E_PAD // B - E // B, B), 1)
    padv = N + (flat % (N_PAD - N))
    er_ref[0, E // B:E_PAD // B, :] = padv
    er_ref[1, E // B:E_PAD // B, :] = padv


@jax.jit
def _pad(ei):
    return pl.pallas_call(
        _pad_body,
        out_shape=jax.ShapeDtypeStruct((2, E_PAD // B, B), jnp.int32),
    )(ei)


def _mmx_body(x_ref, w_ref, u_ref):
    u_ref[...] = jnp.dot(x_ref[...], w_ref[...],
                         preferred_element_type=jnp.float32)


@jax.jit
def _mmx(x, W1):
    return pl.pallas_call(
        _mmx_body,
        out_shape=jax.ShapeDtypeStruct((N, D), jnp.float32),
    )(x, W1)


def _scale_body(u_ref, deg_ref, g_ref):
    dis = _dis_col(deg_ref)
    g_ref[0:N, :] = dis * u_ref[...]
    g_ref[N:N_PAD, :] = jnp.zeros((N_PAD - N, D), jnp.float32)


@jax.jit
def _scale1(u1, deg_p):
    return pl.pallas_call(
        _scale_body,
        out_shape=jax.ShapeDtypeStruct((N_PAD, D), jnp.float32),
    )(u1, deg_p)


def _lin_body(g_ref, deg_ref, b_ref, o_ref):
    dis = _dis_col(deg_ref)
    o_ref[...] = dis * g_ref[0:N, :] + b_ref[...]


@jax.jit
def _lin(g, deg_p, b):
    return pl.pallas_call(
        _lin_body,
        out_shape=jax.ShapeDtypeStruct((N, D), jnp.float32),
    )(g, deg_p, b)


def _mm2_body(acc_ref, lin_ref, deg_ref, w_ref, g2_ref):
    dis = _dis_col(deg_ref)
    acc = acc_ref[0:N, :] + acc_ref[N_PAD:N_PAD + N, :]
    h1 = jnp.maximum(dis * acc + lin_ref[...], 0.0)
    h2 = jnp.dot(h1, w_ref[...], preferred_element_type=jnp.float32)
    g2_ref[0:N, :] = dis * h2
    g2_ref[N:N_PAD, :] = jnp.zeros((N_PAD - N, D), jnp.float32)


@jax.jit
def _mm2(acc1, lin1, deg_p, W2):
    return pl.pallas_call(
        _mm2_body,
        out_shape=jax.ShapeDtypeStruct((N_PAD, D), jnp.float32),
    )(acc1, lin1, deg_p, W2)


def _fin_body(acc_ref, lin_ref, deg_ref, out_ref):
    dis = _dis_col(deg_ref)
    acc = acc_ref[0:N, :] + acc_ref[N_PAD:N_PAD + N, :]
    out_ref[...] = dis * acc + lin_ref[...]


@jax.jit
def _fin(acc2, lin2, deg_p):
    return pl.pallas_call(
        _fin_body,
        out_shape=jax.ShapeDtypeStruct((N, D), jnp.float32),
    )(acc2, lin2, deg_p)


def kernel(x, edge_index, W1, b1, W2, b2):
    er = _pad(edge_index.astype(jnp.int32))

    deg_p = _deg(er)
    u1 = _mmx(x, W1)
    g1 = _scale1(u1, deg_p)
    acc1 = _scatter(g1, er)
    lin1 = _lin(g1, deg_p, b1)
    g2 = _mm2(acc1, lin1, deg_p, W2)
    acc2 = _scatter(g2, er)
    lin2 = _lin(g2, deg_p, b2)
    return _fin(acc2, lin2, deg_p)
